# Initial kernel scaffold; baseline (speedup 1.0000x reference)
import jax
import jax.numpy as jnp
from jax.experimental import pallas as pl


def kernel(z, edge):
    raise NotImplementedError("write your pallas kernel here")


# speedup vs baseline: 1.4561x; 1.4561x over previous
import dataclasses
import functools

import jax
import jax.numpy as jnp
from jax import lax
from jax.experimental import pallas as pl
from jax.experimental.pallas import tpu as pltpu
from jax.experimental.pallas import tpu_sc as plsc

_L = 16
_W = 128
_D = 128


def _normalize_body(z_ref, o_ref):
    x = z_ref[...]
    n = jnp.sqrt(jnp.sum(x * x, axis=1, keepdims=True))
    o_ref[...] = x / jnp.maximum(n, 1e-12)


def _normalize(z):
    return pl.pallas_call(
        _normalize_body,
        out_shape=jax.ShapeDtypeStruct(z.shape, z.dtype),
    )(z)


def _edge_dot(zn, src2d, dst2d):
    n_edges = src2d.shape[1]
    mesh = plsc.VectorSubcoreMesh(core_axis_name="core",
                                  subcore_axis_name="subcore")
    cp = pltpu.CompilerParams()
    if "needs_layout_passes" in pltpu.CompilerParams.__dataclass_fields__:
        cp = dataclasses.replace(cp, needs_layout_passes=False)

    @functools.partial(
        pl.kernel,
        out_type=jax.ShapeDtypeStruct((1, n_edges), jnp.float32),
        mesh=mesh,
        compiler_params=cp,
        scratch_types=[
            pltpu.VMEM((_W, _D), jnp.float32),
            pltpu.VMEM((_W, _D), jnp.float32),
        ],
    )
    def k(zn_hbm, src_hbm, dst_hbm, out_hbm, xbuf, ybuf):
        def body(src_v, dst_v, o_v):
            pltpu.sync_copy(zn_hbm.at[src_v.at[0]], xbuf)
            pltpu.sync_copy(zn_hbm.at[dst_v.at[0]], ybuf)
            for g in range(_W // _L):
                rows = lax.iota(jnp.int32, _L) + (g * _L)

                def col(j, acc):
                    cols = jnp.full((_L,), j, jnp.int32)
                    xv = plsc.load_gather(xbuf, [rows, cols])
                    yv = plsc.load_gather(ybuf, [rows, cols])
                    return acc + xv * yv

                acc = lax.fori_loop(0, _D, col, jnp.zeros((_L,), jnp.float32))
                o_v[0, pl.ds(g * _L, _L)] = 1.0 / (1.0 + jnp.exp(-acc))

        pltpu.emit_pipeline(
            body,
            grid=(n_edges // _W,),
            in_specs=[
                pl.BlockSpec((1, _W), lambda i: (0, i)),
                pl.BlockSpec((1, _W), lambda i: (0, i)),
            ],
            out_specs=[pl.BlockSpec((1, _W), lambda i: (0, i))],
            core_axis_name=("core", "subcore"),
            dimension_semantics=(pltpu.PARALLEL,),
        )(src_hbm, dst_hbm, out_hbm)

    return k(zn, src2d, dst2d)


def kernel(z, edge):
    zn = _normalize(z)
    out = _edge_dot(zn, edge[0:1], edge[1:2])
    return out.reshape(-1)


# speedup vs baseline: 1.7188x; 1.1804x over previous
import dataclasses
import functools

import jax
import jax.numpy as jnp
from jax import lax
from jax.experimental import pallas as pl
from jax.experimental.pallas import tpu as pltpu
from jax.experimental.pallas import tpu_sc as plsc

_L = 16
_D = 128
_CH = 80
_NSC = 32


def _normalize_body(z_ref, o_ref):
    x = z_ref[...]
    n = jnp.sqrt(jnp.sum(x * x, axis=1, keepdims=True))
    o_ref[...] = x / jnp.maximum(n, 1e-12)


def _normalize(z):
    return pl.pallas_call(
        _normalize_body,
        out_shape=jax.ShapeDtypeStruct(z.shape, z.dtype),
    )(z)


def _edge_dot(zn, src, dst):
    n_edges = src.shape[0]
    epw = n_edges // _NSC
    nch = epw // _CH
    mesh = plsc.VectorSubcoreMesh(core_axis_name="core",
                                  subcore_axis_name="subcore")
    cp = pltpu.CompilerParams()
    if "needs_layout_passes" in pltpu.CompilerParams.__dataclass_fields__:
        cp = dataclasses.replace(cp, needs_layout_passes=False)

    @functools.partial(
        pl.kernel,
        out_type=jax.ShapeDtypeStruct((n_edges,), jnp.float32),
        mesh=mesh,
        compiler_params=cp,
        scratch_types=[
            pltpu.VMEM((epw,), jnp.int32),
            pltpu.VMEM((epw,), jnp.int32),
            pltpu.VMEM((_CH, _D), jnp.float32),
            pltpu.VMEM((_CH, _D), jnp.float32),
            pltpu.VMEM((_CH, _D), jnp.float32),
            pltpu.VMEM((_CH, _D), jnp.float32),
            pltpu.VMEM((epw,), jnp.float32),
            pltpu.SemaphoreType.DMA,
            pltpu.SemaphoreType.DMA,
        ],
    )
    def k(zn_hbm, src_hbm, dst_hbm, out_hbm, sidx, didx,
          xb0, xb1, yb0, yb1, out_v, sem0, sem1):
        wid = lax.axis_index("subcore") * 2 + lax.axis_index("core")
        base = wid * epw
        pltpu.sync_copy(src_hbm.at[pl.ds(base, epw)], sidx)
        pltpu.sync_copy(dst_hbm.at[pl.ds(base, epw)], didx)

        xbufs, ybufs, sems = (xb0, xb1), (yb0, yb1), (sem0, sem1)

        def start(c, b):
            pltpu.async_copy(zn_hbm.at[sidx.at[pl.ds(c * _CH, _CH)]],
                             xbufs[b], sems[b])
            pltpu.async_copy(zn_hbm.at[didx.at[pl.ds(c * _CH, _CH)]],
                             ybufs[b], sems[b])

        def drain(b):
            dummy = zn_hbm.at[pl.ds(0, _CH)]
            pltpu.make_async_copy(dummy, xbufs[b], sems[b]).wait()
            pltpu.make_async_copy(dummy, ybufs[b], sems[b]).wait()

        def compute(c, b):
            xbuf, ybuf = xbufs[b], ybufs[b]
            for g in range(_CH // _L):
                rows = lax.iota(jnp.int32, _L) + (g * _L)

                def col(j, carry):
                    acc, cols = carry
                    xv = plsc.load_gather(xbuf, [rows, cols])
                    yv = plsc.load_gather(ybuf, [rows, cols])
                    return acc + xv * yv, cols + 1

                acc, _ = lax.fori_loop(
                    0, _D, col,
                    (jnp.zeros((_L,), jnp.float32),
                     jnp.zeros((_L,), jnp.int32)),
                    unroll=4)
                s = 1.0 / (1.0 + jnp.exp(-acc))
                out_v[pl.ds(c * _CH + g * _L, _L)] = s

        start(0, 0)
        start(1, 1)

        @pl.loop(0, nch - 1, step=2)
        def _(c0):
            for b in range(2):
                c = c0 + b
                drain(b)
                compute(c, b)

                @pl.when(c + 2 < nch)
                def _():
                    start(c + 2, b)

        drain((nch - 1) % 2)
        compute(nch - 1, (nch - 1) % 2)

        pltpu.sync_copy(out_v, out_hbm.at[pl.ds(base, epw)])

    return k(zn, src, dst)


def kernel(z, edge):
    zn = _normalize(z)
    return _edge_dot(zn, edge[0], edge[1])


# speedup vs baseline: 11.2896x; 6.5684x over previous
import dataclasses
import functools

import jax
import jax.numpy as jnp
from jax import lax
from jax.experimental import pallas as pl
from jax.experimental.pallas import tpu as pltpu
from jax.experimental.pallas import tpu_sc as plsc

_L = 16
_D = 128
_DW = 64
_CH = 80
_NSC = 32


def _normalize_body(z_ref, o_ref):
    x = z_ref[...]
    n = jnp.sqrt(jnp.sum(x * x, axis=1, keepdims=True))
    o_ref[...] = (x / jnp.maximum(n, 1e-12)).astype(jnp.bfloat16)


def _normalize_pack(z):
    zn16 = pl.pallas_call(
        _normalize_body,
        out_shape=jax.ShapeDtypeStruct(z.shape, jnp.bfloat16),
    )(z)
    pairs = zn16.reshape(z.shape[0], z.shape[1] // 2, 2)
    return lax.bitcast_convert_type(pairs, jnp.int32)


def _edge_dot(znw, src, dst):
    n_edges = src.shape[0]
    epw = n_edges // _NSC
    nch = epw // _CH
    mesh = plsc.VectorSubcoreMesh(core_axis_name="core",
                                  subcore_axis_name="subcore")
    cp = pltpu.CompilerParams()
    if "needs_layout_passes" in pltpu.CompilerParams.__dataclass_fields__:
        cp = dataclasses.replace(cp, needs_layout_passes=False)
    if "use_tc_tiling_on_sc" in pltpu.CompilerParams.__dataclass_fields__:
        cp = dataclasses.replace(cp, use_tc_tiling_on_sc=False)

    @functools.partial(
        pl.kernel,
        out_type=jax.ShapeDtypeStruct((n_edges,), jnp.float32),
        mesh=mesh,
        compiler_params=cp,
        scratch_types=[
            pltpu.VMEM((epw,), jnp.int32),
            pltpu.VMEM((epw,), jnp.int32),
            pltpu.VMEM((_CH, _DW), jnp.int32),
            pltpu.VMEM((_CH, _DW), jnp.int32),
            pltpu.VMEM((_CH, _DW), jnp.int32),
            pltpu.VMEM((_CH, _DW), jnp.int32),
            pltpu.VMEM((epw,), jnp.float32),
            pltpu.SemaphoreType.DMA,
            pltpu.SemaphoreType.DMA,
        ],
    )
    def k(zn_hbm, src_hbm, dst_hbm, out_hbm, sidx, didx,
          xb0, xb1, yb0, yb1, out_v, sem0, sem1):
        wid = lax.axis_index("subcore") * 2 + lax.axis_index("core")
        base = wid * epw
        pltpu.sync_copy(src_hbm.at[pl.ds(base, epw)], sidx)
        pltpu.sync_copy(dst_hbm.at[pl.ds(base, epw)], didx)

        xbufs, ybufs, sems = (xb0, xb1), (yb0, yb1), (sem0, sem1)

        def start(c, b):
            pltpu.async_copy(zn_hbm.at[sidx.at[pl.ds(c * _CH, _CH)]],
                             xbufs[b], sems[b])
            pltpu.async_copy(zn_hbm.at[didx.at[pl.ds(c * _CH, _CH)]],
                             ybufs[b], sems[b])

        def drain(b):
            dummy = zn_hbm.at[pl.ds(0, _CH)]
            pltpu.make_async_copy(dummy, xbufs[b], sems[b]).wait()
            pltpu.make_async_copy(dummy, ybufs[b], sems[b]).wait()

        lanes = lax.iota(jnp.int32, _L)

        def compute(c, b):
            xbuf, ybuf = xbufs[b], ybufs[b]
            for g in range(_CH // _L):
                rows = lanes + (g * _L)

                def col(j, carry):
                    acc, jv = carry
                    cols = lax.bitwise_xor(jv, lanes)
                    xw = plsc.load_gather(xbuf, [rows, cols])
                    yw = plsc.load_gather(ybuf, [rows, cols])
                    xe, xo = plsc.unpack(plsc.bitcast(xw, jnp.bfloat16),
                                         format=plsc.PackFormat.INTERLEAVED)
                    ye, yo = plsc.unpack(plsc.bitcast(yw, jnp.bfloat16),
                                         format=plsc.PackFormat.INTERLEAVED)
                    return acc + xe * ye + xo * yo, jv + 1

                acc, _ = lax.fori_loop(
                    0, _DW, col,
                    (jnp.zeros((_L,), jnp.float32),
                     jnp.zeros((_L,), jnp.int32)),
                    unroll=4)
                s = 1.0 / (1.0 + jnp.exp(-acc))
                out_v[pl.ds(c * _CH + g * _L, _L)] = s

        start(0, 0)
        start(1, 1)

        @pl.loop(0, nch - 1, step=2)
        def _(c0):
            for b in range(2):
                c = c0 + b
                drain(b)
                compute(c, b)

                @pl.when(c + 2 < nch)
                def _():
                    start(c + 2, b)

        drain((nch - 1) % 2)
        compute(nch - 1, (nch - 1) % 2)

        pltpu.sync_copy(out_v, out_hbm.at[pl.ds(base, epw)])

    return k(znw, src, dst)


def kernel(z, edge):
    znw = _normalize_pack(z)
    return _edge_dot(znw, edge[0], edge[1])
